# Initial kernel scaffold; baseline (speedup 1.0000x reference)
import jax
import jax.numpy as jnp
from jax.experimental import pallas as pl


def kernel(pair_features, atom_features, atom_to_pair, W, b):
    raise NotImplementedError("write your pallas kernel here")


# speedup vs baseline: 3.7705x; 3.7705x over previous
import functools

import jax
import jax.numpy as jnp
import numpy as np
from jax import lax
from jax.experimental import pallas as pl
from jax.experimental.pallas import tpu as pltpu
from jax.experimental.pallas import tpu_sc as plsc

E = 320000
N = 10000
H = 16

NC = 2
NS = 16
NW = NC * NS

SUB = 80
SUBS = 25
BLK = SUB * SUBS
PER_W = E // NW
NBLK = PER_W // BLK
IDX_ROWS_PER_BLK = SUBS
IDX_ROWS_PER_W = PER_W // SUB

_MESH = dict(core_axis_name="c", subcore_axis_name="s")


def _gather_body(table_hbm, idx_hbm, out_hbm, idx_v, rows_v, sem):
    c = lax.axis_index("c")
    s = lax.axis_index("s")
    wid = c * NS + s
    pltpu.sync_copy(idx_hbm.at[wid], idx_v)

    def blk(bi, carry):
        rbase = wid * PER_W + bi * BLK

        def sub(k, carry2):
            pltpu.async_copy(table_hbm.at[idx_v.at[bi * SUBS + k]],
                             rows_v.at[pl.ds(k * SUB, SUB)], sem)
            return carry2

        lax.fori_loop(0, SUBS, sub, 0)
        pltpu.make_async_copy(table_hbm.at[pl.ds(0, BLK)], rows_v, sem).wait()
        pltpu.sync_copy(rows_v, out_hbm.at[pl.ds(rbase, BLK)])
        return carry

    lax.fori_loop(0, NBLK, blk, 0)


def _gather(atom_features, src2d):
    k = functools.partial(
        pl.kernel,
        mesh=plsc.VectorSubcoreMesh(**_MESH),
        out_type=jax.ShapeDtypeStruct((E, H), jnp.float32),
        scratch_types=[
            pltpu.VMEM((IDX_ROWS_PER_W, SUB), jnp.int32),
            pltpu.VMEM((BLK, H), jnp.float32),
            pltpu.SemaphoreType.DMA,
        ],
        compiler_params=pltpu.CompilerParams(use_tc_tiling_on_sc=False),
    )(_gather_body)
    return k(atom_features, src2d)


def _scatter_body(vals_hbm, idx_hbm, zeros_hbm, out_hbm, idx_v, rows_v, acc_sh):
    c = lax.axis_index("c")
    s = lax.axis_index("s")
    wid = c * NS + s

    @pl.when(s == 0)
    def _():
        pltpu.sync_copy(zeros_hbm, acc_sh)

    pltpu.sync_copy(idx_hbm.at[wid], idx_v)
    plsc.subcore_barrier()

    def blk(bi, carry):
        rbase = wid * PER_W + bi * BLK
        pltpu.sync_copy(vals_hbm.at[pl.ds(rbase, BLK)], rows_v)

        def sub(k, carry2):
            pltpu.sync_copy(rows_v.at[pl.ds(k * SUB, SUB)],
                            acc_sh.at[idx_v.at[bi * SUBS + k]], add=True)
            return carry2

        lax.fori_loop(0, SUBS, sub, 0)
        return carry

    lax.fori_loop(0, NBLK, blk, 0)
    plsc.subcore_barrier()

    @pl.when(s == 0)
    def _():
        pltpu.sync_copy(acc_sh, out_hbm.at[c])


def _scatter(vals, dst2d, zeros):
    k = functools.partial(
        pl.kernel,
        mesh=plsc.VectorSubcoreMesh(**_MESH),
        out_type=jax.ShapeDtypeStruct((NC, N, H), jnp.float32),
        scratch_types=[
            pltpu.VMEM((IDX_ROWS_PER_W, SUB), jnp.int32),
            pltpu.VMEM((BLK, H), jnp.float32),
            pltpu.VMEM_SHARED((N, H), jnp.float32),
        ],
        compiler_params=pltpu.CompilerParams(use_tc_tiling_on_sc=False),
    )(_scatter_body)
    return k(vals, dst2d, zeros)


EB = 2000


def _mm_body(pair_ref, g_ref, w_ref, bmt_ref, k_ref, out_ref):
    a = jnp.dot(pair_ref[...], w_ref[...], preferred_element_type=jnp.float32)
    g = g_ref[...]
    gt = jnp.concatenate([g] * H, axis=1)
    m = a * gt
    out = jnp.dot(m, k_ref[...], preferred_element_type=jnp.float32)
    out += jnp.dot(g, bmt_ref[...], preferred_element_type=jnp.float32)
    out_ref[...] = out


def _matmul(pair_features, g, W, b):
    bmt = b.reshape(H, H).T
    kmat = jnp.asarray(_KMAT)
    grid = (E // EB,)
    return pl.pallas_call(
        _mm_body,
        grid=grid,
        in_specs=[
            pl.BlockSpec((EB, H), lambda i: (i, 0)),
            pl.BlockSpec((EB, H), lambda i: (i, 0)),
            pl.BlockSpec((H, H * H), lambda i: (0, 0)),
            pl.BlockSpec((H, H), lambda i: (0, 0)),
            pl.BlockSpec((H * H, H), lambda i: (0, 0)),
        ],
        out_specs=pl.BlockSpec((EB, H), lambda i: (i, 0)),
        out_shape=jax.ShapeDtypeStruct((E, H), jnp.float32),
        compiler_params=pltpu.CompilerParams(
            dimension_semantics=("arbitrary",),
        ),
    )(pair_features, g, W, bmt, kmat)


def _make_kmat():
    k = np.zeros((H * H, H), dtype=np.float32)
    for i in range(H):
        for j in range(H):
            k[i * H + j, i] = 1.0
    return k


_KMAT = _make_kmat()


def _comb_body(parts_ref, out_ref):
    out_ref[...] = parts_ref[0] + parts_ref[1]


def _combine(parts):
    return pl.pallas_call(
        _comb_body,
        out_shape=jax.ShapeDtypeStruct((N, H), jnp.float32),
    )(parts)


def kernel(pair_features, atom_features, atom_to_pair, W, b):
    dst2d = atom_to_pair[:, 0].reshape(NW, IDX_ROWS_PER_W, SUB)
    src2d = atom_to_pair[:, 1].reshape(NW, IDX_ROWS_PER_W, SUB)
    g = _gather(atom_features, src2d)
    out = _matmul(pair_features, g, W, b)
    parts = _scatter(out, dst2d, jnp.zeros((N, H), jnp.float32))
    return _combine(parts)


# speedup vs baseline: 4.6582x; 1.2354x over previous
import functools

import jax
import jax.numpy as jnp
import numpy as np
from jax import lax
from jax.experimental import pallas as pl
from jax.experimental.pallas import tpu as pltpu
from jax.experimental.pallas import tpu_sc as plsc

E = 320000
N = 10000
H = 16

NC = 2
NS = 16
NW = NC * NS

SUB = 80
SUBS = 25
BLK = SUB * SUBS
PER_W = E // NW
NBLK = PER_W // BLK
IDX_ROWS_PER_BLK = SUBS
IDX_ROWS_PER_W = PER_W // SUB

_MESH = dict(core_axis_name="c", subcore_axis_name="s")


def _gather_body(table_hbm, idx_hbm, out_hbm, idx_v, rows_v, sem):
    c = lax.axis_index("c")
    s = lax.axis_index("s")
    wid = c * NS + s
    pltpu.sync_copy(idx_hbm.at[wid], idx_v)

    def blk(bi, carry):
        rbase = wid * PER_W + bi * BLK

        def sub(k, carry2):
            pltpu.async_copy(table_hbm.at[idx_v.at[bi * SUBS + k]],
                             rows_v.at[pl.ds(k * SUB, SUB)], sem)
            return carry2

        lax.fori_loop(0, SUBS, sub, 0)
        pltpu.make_async_copy(table_hbm.at[pl.ds(0, BLK)], rows_v, sem).wait()
        pltpu.sync_copy(rows_v, out_hbm.at[pl.ds(rbase, BLK)])
        return carry

    lax.fori_loop(0, NBLK, blk, 0)


def _gather(atom_features, src2d):
    k = functools.partial(
        pl.kernel,
        mesh=plsc.VectorSubcoreMesh(**_MESH),
        out_type=jax.ShapeDtypeStruct((E, H), jnp.float32),
        scratch_types=[
            pltpu.VMEM((IDX_ROWS_PER_W, SUB), jnp.int32),
            pltpu.VMEM((BLK, H), jnp.float32),
            pltpu.SemaphoreType.DMA,
        ],
        compiler_params=pltpu.CompilerParams(use_tc_tiling_on_sc=False),
    )(_gather_body)
    return k(atom_features, src2d)


def _scatter_body(vals_hbm, idx_hbm, zeros_hbm, out_hbm, idx_v, rows_v, acc_sh):
    c = lax.axis_index("c")
    s = lax.axis_index("s")
    wid = c * NS + s

    @pl.when(s == 0)
    def _():
        pltpu.sync_copy(zeros_hbm, acc_sh)

    pltpu.sync_copy(idx_hbm.at[wid], idx_v)
    plsc.subcore_barrier()

    def blk(bi, carry):
        rbase = wid * PER_W + bi * BLK
        pltpu.sync_copy(vals_hbm.at[pl.ds(rbase, BLK)], rows_v)

        def sub(k, carry2):
            pltpu.sync_copy(rows_v.at[pl.ds(k * SUB, SUB)],
                            acc_sh.at[idx_v.at[bi * SUBS + k]], add=True)
            return carry2

        lax.fori_loop(0, SUBS, sub, 0)
        return carry

    lax.fori_loop(0, NBLK, blk, 0)
    plsc.subcore_barrier()

    @pl.when(s == 0)
    def _():
        pltpu.sync_copy(acc_sh, out_hbm.at[c])


def _scatter(vals, dst2d, zeros):
    k = functools.partial(
        pl.kernel,
        mesh=plsc.VectorSubcoreMesh(**_MESH),
        out_type=jax.ShapeDtypeStruct((NC, N, H), jnp.float32),
        scratch_types=[
            pltpu.VMEM((IDX_ROWS_PER_W, SUB), jnp.int32),
            pltpu.VMEM((BLK, H), jnp.float32),
            pltpu.VMEM_SHARED((N, H), jnp.float32),
        ],
        compiler_params=pltpu.CompilerParams(use_tc_tiling_on_sc=False),
    )(_scatter_body)
    return k(vals, dst2d, zeros)


EB = 4000


def _mm_body(pair_ref, g_ref, w_ref, b_ref, s_ref, k_ref, out_ref):
    g = g_ref[...]
    a = jnp.dot(pair_ref[...], w_ref[...], preferred_element_type=jnp.float32)
    a += b_ref[...]
    gt = jnp.dot(g, s_ref[...], preferred_element_type=jnp.float32)
    out_ref[...] = jnp.dot(a * gt, k_ref[...], preferred_element_type=jnp.float32)


def _matmul(pair_features, g, W, b):
    b2 = b.reshape(1, H * H)
    kmat = jnp.asarray(_KMAT)
    smat = jnp.asarray(_SMAT)
    grid = (E // EB,)
    return pl.pallas_call(
        _mm_body,
        grid=grid,
        in_specs=[
            pl.BlockSpec((EB, H), lambda i: (i, 0)),
            pl.BlockSpec((EB, H), lambda i: (i, 0)),
            pl.BlockSpec((H, H * H), lambda i: (0, 0)),
            pl.BlockSpec((1, H * H), lambda i: (0, 0)),
            pl.BlockSpec((H, H * H), lambda i: (0, 0)),
            pl.BlockSpec((H * H, H), lambda i: (0, 0)),
        ],
        out_specs=pl.BlockSpec((EB, H), lambda i: (i, 0)),
        out_shape=jax.ShapeDtypeStruct((E, H), jnp.float32),
        compiler_params=pltpu.CompilerParams(
            dimension_semantics=("arbitrary",),
        ),
    )(pair_features, g, W, b2, smat, kmat)


def _make_kmat():
    k = np.zeros((H * H, H), dtype=np.float32)
    for i in range(H):
        for j in range(H):
            k[i * H + j, i] = 1.0
    return k


def _make_smat():
    s = np.zeros((H, H * H), dtype=np.float32)
    for i in range(H):
        for j in range(H):
            s[j, i * H + j] = 1.0
    return s


_KMAT = _make_kmat()
_SMAT = _make_smat()


def _comb_body(parts_ref, out_ref):
    out_ref[...] = parts_ref[0] + parts_ref[1]


def _combine(parts):
    return pl.pallas_call(
        _comb_body,
        out_shape=jax.ShapeDtypeStruct((N, H), jnp.float32),
    )(parts)


def kernel(pair_features, atom_features, atom_to_pair, W, b):
    dst2d = atom_to_pair[:, 0].reshape(NW, IDX_ROWS_PER_W, SUB)
    src2d = atom_to_pair[:, 1].reshape(NW, IDX_ROWS_PER_W, SUB)
    g = _gather(atom_features, src2d)
    out = _matmul(pair_features, g, W, b)
    parts = _scatter(out, dst2d, jnp.zeros((N, H), jnp.float32))
    return _combine(parts)


# speedup vs baseline: 6.3420x; 1.3615x over previous
import functools

import jax
import jax.numpy as jnp
import numpy as np
from jax import lax
from jax.experimental import pallas as pl
from jax.experimental.pallas import tpu as pltpu
from jax.experimental.pallas import tpu_sc as plsc

E = 320000
N = 10000
H = 16

NC = 2
NS = 16
NW = NC * NS

NH = 2
EH = E // NH
PER_W = EH // NW
SUB = 40
SUBS = 25
BLK = SUB * SUBS
NBLK = PER_W // BLK
IDXR = PER_W // SUB

_MESH = dict(core_axis_name="c", subcore_axis_name="s")


def _gather_body(table_hbm, idx_hbm, out_hbm, idx_v, rows_v, sem):
    c = lax.axis_index("c")
    s = lax.axis_index("s")
    wid = c * NS + s
    pltpu.sync_copy(idx_hbm.at[wid], idx_v)

    def blk(bi, carry):
        rbase = wid * PER_W + bi * BLK

        def sub(k, carry2):
            pltpu.async_copy(table_hbm.at[idx_v.at[bi * SUBS + k]],
                             rows_v.at[pl.ds(k * SUB, SUB)], sem)
            return carry2

        lax.fori_loop(0, SUBS, sub, 0)
        pltpu.make_async_copy(table_hbm.at[pl.ds(0, BLK)], rows_v, sem).wait()
        pltpu.sync_copy(rows_v, out_hbm.at[pl.ds(rbase, BLK)])
        return carry

    lax.fori_loop(0, NBLK, blk, 0)


def _gather(atom_features, src3d):
    k = functools.partial(
        pl.kernel,
        mesh=plsc.VectorSubcoreMesh(**_MESH),
        out_type=jax.ShapeDtypeStruct((EH, H), jnp.float32),
        scratch_types=[
            pltpu.VMEM((IDXR, SUB), jnp.int32),
            pltpu.VMEM((BLK, H), jnp.float32),
            pltpu.SemaphoreType.DMA,
        ],
        compiler_params=pltpu.CompilerParams(use_tc_tiling_on_sc=False),
    )(_gather_body)
    return k(atom_features, src3d)


def _scatter_body(vals_hbm, idx_hbm, zeros_hbm, out_hbm, idx_v, rows_v, acc_sh):
    c = lax.axis_index("c")
    s = lax.axis_index("s")
    wid = c * NS + s

    @pl.when(s == 0)
    def _():
        pltpu.sync_copy(zeros_hbm, acc_sh)

    pltpu.sync_copy(idx_hbm.at[wid], idx_v)
    plsc.subcore_barrier()

    def blk(bi, carry):
        rbase = wid * PER_W + bi * BLK
        pltpu.sync_copy(vals_hbm.at[pl.ds(rbase, BLK)], rows_v)

        def sub(k, carry2):
            pltpu.sync_copy(rows_v.at[pl.ds(k * SUB, SUB)],
                            acc_sh.at[idx_v.at[bi * SUBS + k]], add=True)
            return carry2

        lax.fori_loop(0, SUBS, sub, 0)
        return carry

    lax.fori_loop(0, NBLK, blk, 0)
    plsc.subcore_barrier()

    @pl.when(s == 0)
    def _():
        pltpu.sync_copy(acc_sh, out_hbm.at[c])


def _scatter(vals, dst3d, zeros):
    k = functools.partial(
        pl.kernel,
        mesh=plsc.VectorSubcoreMesh(**_MESH),
        out_type=jax.ShapeDtypeStruct((NC, N, H), jnp.float32),
        scratch_types=[
            pltpu.VMEM((IDXR, SUB), jnp.int32),
            pltpu.VMEM((BLK, H), jnp.float32),
            pltpu.VMEM_SHARED((N, H), jnp.float32),
        ],
        compiler_params=pltpu.CompilerParams(use_tc_tiling_on_sc=False),
    )(_scatter_body)
    return k(vals, dst3d, zeros)


PK = 8
EP = EH // PK
EBP = 2000


def _mm_body(pair_ref, g_ref, wbig_ref, bmtbig_ref, sbig_ref, kbig_ref,
             out_ref):
    pair_bf = pair_ref[...].astype(jnp.bfloat16)
    g_bf = g_ref[...].astype(jnp.bfloat16)
    a = jnp.dot(pair_bf, wbig_ref[...],
                preferred_element_type=jnp.float32).astype(jnp.bfloat16)
    gt = jnp.dot(g_bf, sbig_ref[...],
                 preferred_element_type=jnp.float32).astype(jnp.bfloat16)
    m = a * gt
    out = jnp.dot(m, kbig_ref[...], preferred_element_type=jnp.float32)
    out += jnp.dot(g_bf, bmtbig_ref[...], preferred_element_type=jnp.float32)
    out_ref[...] = out


def _matmul(pair128, g128, wbig, bmtbig, sbig, kbig):
    grid = (EP // EBP,)
    return pl.pallas_call(
        _mm_body,
        grid=grid,
        in_specs=[
            pl.BlockSpec((EBP, PK * H), lambda i: (i, 0)),
            pl.BlockSpec((EBP, PK * H), lambda i: (i, 0)),
            pl.BlockSpec((PK * H, PK * H * H), lambda i: (0, 0)),
            pl.BlockSpec((PK * H, PK * H), lambda i: (0, 0)),
            pl.BlockSpec((PK * H, PK * H * H), lambda i: (0, 0)),
            pl.BlockSpec((PK * H * H, PK * H), lambda i: (0, 0)),
        ],
        out_specs=pl.BlockSpec((EBP, PK * H), lambda i: (i, 0)),
        out_shape=jax.ShapeDtypeStruct((EP, PK * H), jnp.float32),
        compiler_params=pltpu.CompilerParams(
            dimension_semantics=("arbitrary",),
        ),
    )(pair128, g128, wbig, bmtbig, sbig, kbig)


def _weights(W, b):
    eye8 = jnp.eye(PK, dtype=jnp.float32)
    wbig = jnp.einsum("uv,pc->upvc", eye8, W).reshape(
        PK * H, PK * H * H).astype(jnp.bfloat16)
    sbig = jnp.einsum("uv,pc->upvc", eye8, jnp.asarray(_SMAT)).reshape(
        PK * H, PK * H * H).astype(jnp.bfloat16)
    kbig = jnp.einsum("uv,pc->upvc", eye8, jnp.asarray(_KMAT)).reshape(
        PK * H * H, PK * H).astype(jnp.bfloat16)
    bmt = b.reshape(H, H).T
    bmtbig = jnp.einsum("uv,ji->ujvi", eye8, bmt).reshape(
        PK * H, PK * H).astype(jnp.bfloat16)
    return wbig, bmtbig, sbig, kbig


def _make_kmat():
    k = np.zeros((H * H, H), dtype=np.float32)
    for i in range(H):
        for j in range(H):
            k[i * H + j, i] = 1.0
    return k


def _make_smat():
    s = np.zeros((H, H * H), dtype=np.float32)
    for i in range(H):
        for j in range(H):
            s[j, i * H + j] = 1.0
    return s


_KMAT = _make_kmat()
_SMAT = _make_smat()


def _comb_body(p0_ref, p1_ref, out_ref):
    out_ref[...] = (p0_ref[0] + p0_ref[1]) + (p1_ref[0] + p1_ref[1])


def _combine(parts0, parts1):
    return pl.pallas_call(
        _comb_body,
        out_shape=jax.ShapeDtypeStruct((N, H), jnp.float32),
    )(parts0, parts1)


def kernel(pair_features, atom_features, atom_to_pair, W, b):
    dst = atom_to_pair[:, 0]
    src = atom_to_pair[:, 1]
    wbig, bmtbig, sbig, kbig = _weights(W, b)
    zeros = jnp.zeros((N, H), jnp.float32)
    pair128 = (pair_features.T.reshape(H, E // PK, PK)
               .transpose(1, 2, 0).reshape(E // PK, PK * H))
    parts = []
    for h in range(NH):
        sl = slice(h * EH, (h + 1) * EH)
        src3d = src[sl].reshape(NW, IDXR, SUB)
        dst3d = dst[sl].reshape(NW, IDXR, SUB)
        g = _gather(atom_features, src3d)
        out128 = _matmul(pair128[h * EP:(h + 1) * EP], g.reshape(EP, PK * H),
                         wbig, bmtbig, sbig, kbig)
        parts.append(_scatter(out128.reshape(EH, H), dst3d, zeros))
    return _combine(*parts)
